# baseline (speedup 1.0000x reference)
import functools

import jax
import jax.numpy as jnp
from jax import lax
from jax.experimental import pallas as pl
from jax.experimental.pallas import tpu as pltpu
from jax.experimental.pallas import tpu_sc as plsc

N = 10000
E = 160000
IN_DIM = 256
OUT_DIM = 256

NC = 2
NS = 16
HALF = OUT_DIM // 2
EPT = E // NS
CHUNK = 128
EPT_PAD = 10240
NCHUNK = EPT_PAD // CHUNK
HNCHUNK = NCHUNK // 2
PAD_N = 10240
ROWS_PT = PAD_N // NS
TAIL = N - (NS - 1) * ROWS_PT
REST = ROWS_PT - TAIL


def _mm_body(x_ref, wt_ref, b_ref, o_ref):
    ht = jnp.dot(x_ref[...], wt_ref[...], preferred_element_type=jnp.float32)
    ht = ht + b_ref[...]
    o_ref[0, :, :] = ht[:, :HALF]
    o_ref[1, :, :] = ht[:, HALF:]


def _matmul_split(x, Wt, b2):
    TN = 1000
    grid = (N // TN,)
    return pl.pallas_call(
        _mm_body,
        grid=grid,
        in_specs=[
            pl.BlockSpec((TN, IN_DIM), lambda i: (i, 0)),
            pl.BlockSpec((IN_DIM, OUT_DIM), lambda i: (0, 0)),
            pl.BlockSpec((1, OUT_DIM), lambda i: (0, 0)),
        ],
        out_specs=pl.BlockSpec((2, TN, HALF), lambda i: (0, i, 0)),
        out_shape=jax.ShapeDtypeStruct((2, N, HALF), jnp.float32),
    )(x, Wt, b2)


def _sc_body(hcat_hbm, srcb_hbm, dst_hbm, zer_hbm, out_hbm,
             src_v, dst_v, rows_a, rows_b, acc_sh,
             sem_a, sem_b, sem_sa, sem_sb):
    c = lax.axis_index("c")
    s = lax.axis_index("s")

    pltpu.sync_copy(zer_hbm.at[pl.ds(s * ROWS_PT, ROWS_PT)],
                    acc_sh.at[pl.ds(s * ROWS_PT, ROWS_PT)])
    plsc.subcore_barrier()

    for h in range(2):
        pltpu.sync_copy(srcb_hbm.at[c, s, pl.ds(h * HNCHUNK, HNCHUNK)], src_v)
        pltpu.sync_copy(dst_hbm.at[s, pl.ds(h * HNCHUNK, HNCHUNK)], dst_v)
        pltpu.async_copy(hcat_hbm.at[src_v.at[0]], rows_a, sem_a)
        pltpu.async_copy(hcat_hbm.at[src_v.at[1]], rows_b, sem_b)

        @pl.loop(0, HNCHUNK // 2)
        def _(j):
            ja = 2 * j
            pltpu.make_async_copy(hcat_hbm.at[src_v.at[ja]], rows_a,
                                  sem_a).wait()
            pltpu.async_copy(rows_a, acc_sh.at[dst_v.at[ja]], sem_sa,
                             add=True)

            pltpu.make_async_copy(hcat_hbm.at[src_v.at[ja + 1]], rows_b,
                                  sem_b).wait()
            pltpu.async_copy(rows_b, acc_sh.at[dst_v.at[ja + 1]], sem_sb,
                             add=True)

            pltpu.make_async_copy(rows_a, acc_sh.at[dst_v.at[ja]],
                                  sem_sa).wait()

            @pl.when(j < HNCHUNK // 2 - 1)
            def _():
                pltpu.async_copy(hcat_hbm.at[src_v.at[ja + 2]], rows_a, sem_a)

            pltpu.make_async_copy(rows_b, acc_sh.at[dst_v.at[ja + 1]],
                                  sem_sb).wait()

            @pl.when(j < HNCHUNK // 2 - 1)
            def _():
                pltpu.async_copy(hcat_hbm.at[src_v.at[ja + 3]], rows_b, sem_b)

    plsc.subcore_barrier()
    pltpu.sync_copy(acc_sh.at[pl.ds(s * ROWS_PT, ROWS_PT)],
                    out_hbm.at[c, pl.ds(s * ROWS_PT, ROWS_PT)])


def _sc_aggregate(hcat, srcb, dst3, zer):
    mesh = plsc.VectorSubcoreMesh(core_axis_name="c", subcore_axis_name="s")
    run = pl.kernel(
        _sc_body,
        out_type=jax.ShapeDtypeStruct((2, PAD_N, HALF), jnp.float32),
        mesh=mesh,
        scratch_types=[
            pltpu.VMEM((HNCHUNK, CHUNK), jnp.int32),
            pltpu.VMEM((HNCHUNK, CHUNK), jnp.int32),
            pltpu.VMEM((CHUNK, HALF), jnp.float32),
            pltpu.VMEM((CHUNK, HALF), jnp.float32),
            pltpu.VMEM_SHARED((PAD_N, HALF), jnp.float32),
            pltpu.SemaphoreType.DMA,
            pltpu.SemaphoreType.DMA,
            pltpu.SemaphoreType.DMA,
            pltpu.SemaphoreType.DMA,
        ],
    )
    return run(hcat, srcb, dst3, zer)


def kernel(x, edge_index, W, b):
    Wt = W.T
    b2 = b.reshape(1, OUT_DIM)
    h2 = _matmul_split(x, Wt, b2)
    hcat = h2.reshape(2 * N, HALF)

    src = edge_index[0].astype(jnp.int32).reshape(NS, EPT)
    dst = edge_index[1].astype(jnp.int32).reshape(NS, EPT)
    npad = EPT_PAD - EPT
    pad_src = jnp.broadcast_to((jnp.arange(npad, dtype=jnp.int32) * 41) % N,
                               (NS, npad))
    pad_dst = jnp.broadcast_to(jnp.arange(N, N + npad, dtype=jnp.int32),
                               (NS, npad))
    src_p = jnp.concatenate([src, pad_src], axis=1).reshape(NS, NCHUNK, CHUNK)
    dst3 = jnp.concatenate([dst, pad_dst], axis=1).reshape(NS, NCHUNK, CHUNK)
    srcb = src_p[None] + jnp.array([0, N], jnp.int32).reshape(2, 1, 1, 1)
    zer = jnp.zeros((PAD_N, HALF), jnp.float32)

    out2 = _sc_aggregate(hcat, srcb, dst3, zer)
    return jnp.concatenate([out2[0, :N], out2[1, :N]], axis=1)


# speedup vs baseline: 1.2590x; 1.2590x over previous
import functools

import jax
import jax.numpy as jnp
from jax import lax
from jax.experimental import pallas as pl
from jax.experimental.pallas import tpu as pltpu
from jax.experimental.pallas import tpu_sc as plsc

N = 10000
E = 160000
IN_DIM = 256
OUT_DIM = 256

NC = 2
NS = 16
HALF = OUT_DIM // 2
EPT = E // NS
CHUNK = 128
EPT_PAD = 10240
NCHUNK = EPT_PAD // CHUNK
HNCHUNK = NCHUNK // 2
PAD_N = 10240
ROWS_PT = PAD_N // NS


def _mm_body(x_ref, wt_ref, b_ref, o_ref):
    ht = jnp.dot(x_ref[...], wt_ref[...], preferred_element_type=jnp.float32)
    ht = ht + b_ref[...]
    o_ref[0, :, :] = ht[:, :HALF]
    o_ref[1, :, :] = ht[:, HALF:]


def _matmul_split(x, Wt, b2):
    TN = 1000
    grid = (N // TN,)
    return pl.pallas_call(
        _mm_body,
        grid=grid,
        in_specs=[
            pl.BlockSpec((TN, IN_DIM), lambda i: (i, 0)),
            pl.BlockSpec((IN_DIM, OUT_DIM), lambda i: (0, 0)),
            pl.BlockSpec((1, OUT_DIM), lambda i: (0, 0)),
        ],
        out_specs=pl.BlockSpec((2, TN, HALF), lambda i: (0, i, 0)),
        out_shape=jax.ShapeDtypeStruct((2, N, HALF), jnp.float32),
    )(x, Wt, b2)


def _sc_body(h2_hbm, src_hbm, dst_hbm, out_hbm,
             src_v, dst_v, rows_a, rows_b, acc_sh, sem_a, sem_b):
    c = lax.axis_index("c")
    s = lax.axis_index("s")
    hc = h2_hbm.at[c]

    rows_a[...] = jnp.zeros((CHUNK, HALF), jnp.float32)
    for k in range(ROWS_PT // CHUNK):
        pltpu.sync_copy(rows_a,
                        acc_sh.at[pl.ds(s * ROWS_PT + k * CHUNK, CHUNK)])
    plsc.subcore_barrier()

    for h in range(2):
        pltpu.sync_copy(src_hbm.at[s, pl.ds(h * HNCHUNK, HNCHUNK)], src_v)
        pltpu.sync_copy(dst_hbm.at[s, pl.ds(h * HNCHUNK, HNCHUNK)], dst_v)
        pltpu.async_copy(hc.at[src_v.at[0]], rows_a, sem_a)
        pltpu.async_copy(hc.at[src_v.at[1]], rows_b, sem_b)

        @pl.loop(0, HNCHUNK // 2)
        def _(j):
            ja = 2 * j
            pltpu.make_async_copy(hc.at[src_v.at[ja]], rows_a,
                                  sem_a).wait()
            pltpu.sync_copy(rows_a, acc_sh.at[dst_v.at[ja]], add=True)

            @pl.when(j < HNCHUNK // 2 - 1)
            def _():
                pltpu.async_copy(hc.at[src_v.at[ja + 2]], rows_a, sem_a)

            pltpu.make_async_copy(hc.at[src_v.at[ja + 1]], rows_b,
                                  sem_b).wait()
            pltpu.sync_copy(rows_b, acc_sh.at[dst_v.at[ja + 1]], add=True)

            @pl.when(j < HNCHUNK // 2 - 1)
            def _():
                pltpu.async_copy(hc.at[src_v.at[ja + 3]], rows_b, sem_b)

    plsc.subcore_barrier()
    pltpu.sync_copy(acc_sh.at[pl.ds(s * ROWS_PT, ROWS_PT)],
                    out_hbm.at[c, pl.ds(s * ROWS_PT, ROWS_PT)])


def _sc_aggregate(h2, src_p, dst3):
    mesh = plsc.VectorSubcoreMesh(core_axis_name="c", subcore_axis_name="s")
    run = pl.kernel(
        _sc_body,
        out_type=jax.ShapeDtypeStruct((2, PAD_N, HALF), jnp.float32),
        mesh=mesh,
        scratch_types=[
            pltpu.VMEM((HNCHUNK, CHUNK), jnp.int32),
            pltpu.VMEM((HNCHUNK, CHUNK), jnp.int32),
            pltpu.VMEM((CHUNK, HALF), jnp.float32),
            pltpu.VMEM((CHUNK, HALF), jnp.float32),
            pltpu.VMEM_SHARED((PAD_N, HALF), jnp.float32),
            pltpu.SemaphoreType.DMA,
            pltpu.SemaphoreType.DMA,
        ],
    )
    return run(h2, src_p, dst3)


def kernel(x, edge_index, W, b):
    Wt = W.T
    b2 = b.reshape(1, OUT_DIM)
    h2 = _matmul_split(x, Wt, b2)

    src = edge_index[0].astype(jnp.int32).reshape(NS, EPT)
    dst = edge_index[1].astype(jnp.int32).reshape(NS, EPT)
    npad = EPT_PAD - EPT
    pad_src = jnp.broadcast_to((jnp.arange(npad, dtype=jnp.int32) * 41) % N,
                               (NS, npad))
    pad_dst = jnp.broadcast_to(jnp.arange(N, N + npad, dtype=jnp.int32),
                               (NS, npad))
    src_p = jnp.concatenate([src, pad_src], axis=1).reshape(NS, NCHUNK, CHUNK)
    dst3 = jnp.concatenate([dst, pad_dst], axis=1).reshape(NS, NCHUNK, CHUNK)

    out2 = _sc_aggregate(h2, src_p, dst3)
    return jnp.concatenate([out2[0, :N], out2[1, :N]], axis=1)


# speedup vs baseline: 1.2609x; 1.0015x over previous
import functools

import jax
import jax.numpy as jnp
from jax import lax
from jax.experimental import pallas as pl
from jax.experimental.pallas import tpu as pltpu
from jax.experimental.pallas import tpu_sc as plsc

N = 10000
E = 160000
IN_DIM = 256
OUT_DIM = 256

NC = 2
NS = 16
HALF = OUT_DIM // 2
EPT = E // NS
CHUNK = 128
EPT_PAD = 10240
NCHUNK = EPT_PAD // CHUNK
HNCHUNK = NCHUNK // 2
PAD_N = 10240
ROWS_PT = PAD_N // NS


def _mm_body(x_ref, wt_ref, b_ref, o_ref):
    ht = jnp.dot(x_ref[...].astype(jnp.bfloat16),
                 wt_ref[...].astype(jnp.bfloat16),
                 preferred_element_type=jnp.float32)
    ht = ht + b_ref[...]
    o_ref[0, :, :] = ht[:, :HALF]
    o_ref[1, :, :] = ht[:, HALF:]


def _matmul_split(x, Wt, b2):
    TN = 1000
    grid = (N // TN,)
    return pl.pallas_call(
        _mm_body,
        grid=grid,
        in_specs=[
            pl.BlockSpec((TN, IN_DIM), lambda i: (i, 0)),
            pl.BlockSpec((IN_DIM, OUT_DIM), lambda i: (0, 0)),
            pl.BlockSpec((1, OUT_DIM), lambda i: (0, 0)),
        ],
        out_specs=pl.BlockSpec((2, TN, HALF), lambda i: (0, i, 0)),
        out_shape=jax.ShapeDtypeStruct((2, N, HALF), jnp.float32),
    )(x, Wt, b2)


def _sc_body(h2_hbm, src_hbm, dst_hbm, out_hbm,
             src_v, dst_v, rows_a, rows_b, acc_sh, sem_a, sem_b):
    c = lax.axis_index("c")
    s = lax.axis_index("s")
    hc = h2_hbm.at[c]

    rows_a[...] = jnp.zeros((CHUNK, HALF), jnp.float32)
    for k in range(ROWS_PT // CHUNK):
        pltpu.sync_copy(rows_a,
                        acc_sh.at[pl.ds(s * ROWS_PT + k * CHUNK, CHUNK)])
    plsc.subcore_barrier()

    for h in range(2):
        pltpu.sync_copy(src_hbm.at[s, pl.ds(h * HNCHUNK, HNCHUNK)], src_v)
        pltpu.sync_copy(dst_hbm.at[s, pl.ds(h * HNCHUNK, HNCHUNK)], dst_v)
        pltpu.async_copy(hc.at[src_v.at[0]], rows_a, sem_a)
        pltpu.async_copy(hc.at[src_v.at[1]], rows_b, sem_b)

        @pl.loop(0, HNCHUNK // 2)
        def _(j):
            ja = 2 * j
            pltpu.make_async_copy(hc.at[src_v.at[ja]], rows_a,
                                  sem_a).wait()
            pltpu.sync_copy(rows_a, acc_sh.at[dst_v.at[ja]], add=True)

            @pl.when(j < HNCHUNK // 2 - 1)
            def _():
                pltpu.async_copy(hc.at[src_v.at[ja + 2]], rows_a, sem_a)

            pltpu.make_async_copy(hc.at[src_v.at[ja + 1]], rows_b,
                                  sem_b).wait()
            pltpu.sync_copy(rows_b, acc_sh.at[dst_v.at[ja + 1]], add=True)

            @pl.when(j < HNCHUNK // 2 - 1)
            def _():
                pltpu.async_copy(hc.at[src_v.at[ja + 3]], rows_b, sem_b)

    plsc.subcore_barrier()
    pltpu.sync_copy(acc_sh.at[pl.ds(s * ROWS_PT, ROWS_PT)],
                    out_hbm.at[c, pl.ds(s * ROWS_PT, ROWS_PT)])


def _sc_aggregate(h2, src_p, dst3):
    mesh = plsc.VectorSubcoreMesh(core_axis_name="c", subcore_axis_name="s")
    run = pl.kernel(
        _sc_body,
        out_type=jax.ShapeDtypeStruct((2, PAD_N, HALF), jnp.float32),
        mesh=mesh,
        scratch_types=[
            pltpu.VMEM((HNCHUNK, CHUNK), jnp.int32),
            pltpu.VMEM((HNCHUNK, CHUNK), jnp.int32),
            pltpu.VMEM((CHUNK, HALF), jnp.float32),
            pltpu.VMEM((CHUNK, HALF), jnp.float32),
            pltpu.VMEM_SHARED((PAD_N, HALF), jnp.float32),
            pltpu.SemaphoreType.DMA,
            pltpu.SemaphoreType.DMA,
        ],
    )
    return run(h2, src_p, dst3)


def kernel(x, edge_index, W, b):
    Wt = W.T
    b2 = b.reshape(1, OUT_DIM)
    h2 = _matmul_split(x, Wt, b2)

    src = edge_index[0].astype(jnp.int32).reshape(NS, EPT)
    dst = edge_index[1].astype(jnp.int32).reshape(NS, EPT)
    npad = EPT_PAD - EPT
    pad_src = jnp.broadcast_to((jnp.arange(npad, dtype=jnp.int32) * 41) % N,
                               (NS, npad))
    pad_dst = jnp.broadcast_to(jnp.arange(N, N + npad, dtype=jnp.int32),
                               (NS, npad))
    src_p = jnp.concatenate([src, pad_src], axis=1).reshape(NS, NCHUNK, CHUNK)
    dst3 = jnp.concatenate([dst, pad_dst], axis=1).reshape(NS, NCHUNK, CHUNK)

    out2 = _sc_aggregate(h2, src_p, dst3)
    return jnp.concatenate([out2[0, :N], out2[1, :N]], axis=1)
